# baseline (speedup 1.0000x reference)
import functools

import jax
import jax.numpy as jnp
from jax import lax
from jax.experimental import pallas as pl
from jax.experimental.pallas import tpu as pltpu
from jax.experimental.pallas import tpu_sc as plsc

B, C, D = 1024, 512, 64
NROW = B * C * D // 8192
RB = 256
NC, NS, L = 2, 16, 16
NW = NC * NS
BPW = B // NW
NPAIR = B * C // 2


def _cls_body(a_ref, b_ref, cls_ref):
    a = a_ref[...]
    out = jnp.dot(a * a, b_ref[...], preferred_element_type=jnp.float32,
                  precision=lax.Precision.HIGHEST)
    cls_ref[...] = jnp.sqrt(out)


def _classes(xa, bmat):
    return pl.pallas_call(
        _cls_body,
        grid=(NROW // RB,),
        in_specs=[
            pl.BlockSpec((RB, 8192), lambda i: (i, 0)),
            pl.BlockSpec((8192, 128), lambda i: (0, 0)),
        ],
        out_specs=pl.BlockSpec((RB, 128), lambda i: (i, 0)),
        out_shape=jax.ShapeDtypeStruct((NROW, 128), jnp.float32),
    )(xa, bmat)


_mesh = plsc.VectorSubcoreMesh(core_axis_name="c", subcore_axis_name="s")


@functools.partial(
    pl.kernel,
    out_type=[
        jax.ShapeDtypeStruct((B,), jnp.int32),
        jax.ShapeDtypeStruct((B * D,), jnp.float32),
    ],
    mesh=_mesh,
    compiler_params=pltpu.CompilerParams(needs_layout_passes=False),
    scratch_types=[
        pltpu.VMEM((BPW * C,), jnp.float32),
        pltpu.VMEM((BPW,), jnp.int32),
        pltpu.VMEM((BPW, 128), jnp.float32),
        pltpu.VMEM((BPW * D,), jnp.float32),
        pltpu.SemaphoreType.DMA,
    ],
)
def _sc_pick(cls_hbm, x2_hbm, idx_hbm, masked_hbm,
             cls_v, idx_v, pairs_v, out_v, sem):
    wid = lax.axis_index("s") * NC + lax.axis_index("c")
    base = wid * BPW
    pltpu.sync_copy(cls_hbm.at[pl.ds(base * C, BPW * C)], cls_v)
    lane = lax.broadcasted_iota(jnp.int32, (L,), 0)
    for g in range(BPW // L):
        rids = g * L + lane

        def body(c, carry):
            vmax, varg = carry
            v = plsc.load_gather(cls_v, [rids * C + c])
            take = v > vmax
            return jnp.where(take, v, vmax), jnp.where(take, c, varg)

        _, varg = lax.fori_loop(
            0, C, body,
            (jnp.full((L,), -1.0, jnp.float32), jnp.zeros((L,), jnp.int32)),
        )
        idx_v[pl.ds(g * L, L)] = varg
        cap = (base + rids) * C + varg
        pltpu.async_copy(x2_hbm.at[cap >> 1], pairs_v.at[pl.ds(g * L, L)],
                         sem).wait()
        half = (cap & 1) * D
        for d in range(D):
            val = plsc.load_gather(pairs_v, [rids, half + d])
            plsc.store_scatter(out_v, [rids * D + d], val)
    pltpu.sync_copy(idx_v, idx_hbm.at[pl.ds(base, BPW)])
    pltpu.sync_copy(out_v, masked_hbm.at[pl.ds(base * D, BPW * D)])


def kernel(x):
    xa = x.reshape(NROW, 8192)
    bmat = (jnp.arange(8192, dtype=jnp.int32)[:, None] // D
            == jnp.arange(128, dtype=jnp.int32)[None, :]).astype(jnp.float32)
    cls = _classes(xa, bmat)
    idx, masked = _sc_pick(cls.reshape(B * C), x.reshape(NPAIR, 128))
    return masked.reshape(B, 1, D), idx, cls.reshape(B, C)


# speedup vs baseline: 1.7369x; 1.7369x over previous
import functools

import jax
import jax.numpy as jnp
from jax import lax
from jax.experimental import pallas as pl
from jax.experimental.pallas import tpu as pltpu
from jax.experimental.pallas import tpu_sc as plsc

B, C, D = 1024, 512, 64
BBLK = 8
NC, NS, L = 2, 16, 16
NW = NC * NS
BPW = B // NW
NTILE = B * C // 8


def _cls_body(x_ref, cls_ref):
    xb = x_ref[...]
    s = jnp.sum(xb * xb, axis=2)
    cls_ref[...] = jnp.sqrt(s).reshape(BBLK * 4, 128)


def _classes(x):
    return pl.pallas_call(
        _cls_body,
        grid=(B // BBLK,),
        in_specs=[pl.BlockSpec((BBLK, C, D), lambda i: (i, 0, 0))],
        out_specs=pl.BlockSpec((BBLK * 4, 128), lambda i: (i, 0)),
        out_shape=jax.ShapeDtypeStruct((B * C // 128, 128), jnp.float32),
    )(x)


_mesh = plsc.VectorSubcoreMesh(core_axis_name="c", subcore_axis_name="s")


@functools.partial(
    pl.kernel,
    out_type=[
        jax.ShapeDtypeStruct((B,), jnp.int32),
        jax.ShapeDtypeStruct((B * D,), jnp.float32),
    ],
    mesh=_mesh,
    compiler_params=pltpu.CompilerParams(needs_layout_passes=False),
    scratch_types=[
        pltpu.VMEM((BPW * C,), jnp.float32),
        pltpu.VMEM((BPW,), jnp.int32),
        pltpu.VMEM((8, D), jnp.float32),
        pltpu.VMEM((BPW * D,), jnp.float32),
        pltpu.SemaphoreType.DMA,
    ],
)
def _sc_pick(cls_hbm, x46_hbm, idx_hbm, masked_hbm,
             cls_v, idx_v, tile_v, out_v, sem):
    wid = lax.axis_index("s") * NC + lax.axis_index("c")
    base = wid * BPW
    pltpu.sync_copy(cls_hbm.at[pl.ds(base * C, BPW * C)], cls_v)
    lane = lax.broadcasted_iota(jnp.int32, (L,), 0)
    for g in range(BPW // L):

        def row_body(jj, acc):
            j = g * L + jj

            def chunk(ci, carry):
                vmax, varg = carry
                v = cls_v[pl.ds(j * C + ci * L, L)]
                take = v > vmax
                return (jnp.where(take, v, vmax),
                        jnp.where(take, ci * L + lane, varg))

            vmax, varg = lax.fori_loop(
                0, C // L, chunk,
                (jnp.full((L,), -1.0, jnp.float32),
                 jnp.zeros((L,), jnp.int32)),
            )
            m = jnp.max(vmax)
            c = jnp.min(jnp.where(vmax == m, varg, C))
            t = ((base + j) * C + c) >> 3
            pltpu.sync_copy(x46_hbm.at[t], tile_v)
            k = c & 7
            for mm in range(D // L):
                out_v[pl.ds(j * D + mm * L, L)] = tile_v[k, pl.ds(mm * L, L)]
            return jnp.where(lane == jj, c, acc)

        acc = lax.fori_loop(0, L, row_body, jnp.zeros((L,), jnp.int32))
        idx_v[pl.ds(g * L, L)] = acc
    pltpu.sync_copy(idx_v, idx_hbm.at[pl.ds(base, BPW)])
    pltpu.sync_copy(out_v, masked_hbm.at[pl.ds(base * D, BPW * D)])


def kernel(x):
    cls = _classes(x)
    idx, masked = _sc_pick(cls.reshape(B * C), x.reshape(NTILE, 8, D))
    return masked.reshape(B, 1, D), idx, cls.reshape(B, C)


# speedup vs baseline: 2.0081x; 1.1562x over previous
import functools

import jax
import jax.numpy as jnp
from jax import lax
from jax.experimental import pallas as pl
from jax.experimental.pallas import tpu as pltpu
from jax.experimental.pallas import tpu_sc as plsc

B, C, D = 1024, 512, 64
BBLK = 32
NC, NS, L = 2, 16, 16
NW = NC * NS
BPW = B // NW
NTILE = B * C // 8


def _cls_body(x_ref, cls_ref):
    xb = x_ref[...]
    s = jnp.sum(xb * xb, axis=2)
    cls_ref[...] = jnp.sqrt(s).reshape(BBLK * 4, 128)


def _classes(x):
    return pl.pallas_call(
        _cls_body,
        grid=(B // BBLK,),
        in_specs=[pl.BlockSpec((BBLK, C, D), lambda i: (i, 0, 0))],
        out_specs=pl.BlockSpec((BBLK * 4, 128), lambda i: (i, 0)),
        out_shape=jax.ShapeDtypeStruct((B * C // 128, 128), jnp.float32),
    )(x)


_mesh = plsc.VectorSubcoreMesh(core_axis_name="c", subcore_axis_name="s")


@functools.partial(
    pl.kernel,
    out_type=[
        jax.ShapeDtypeStruct((B,), jnp.int32),
        jax.ShapeDtypeStruct((B * D,), jnp.float32),
    ],
    mesh=_mesh,
    compiler_params=pltpu.CompilerParams(needs_layout_passes=False),
    scratch_types=[
        pltpu.VMEM((BPW * C,), jnp.float32),
        pltpu.VMEM((BPW,), jnp.int32),
        pltpu.VMEM((8, D), jnp.float32),
        pltpu.VMEM((BPW * D,), jnp.float32),
        pltpu.SemaphoreType.DMA,
    ],
)
def _sc_pick(cls_hbm, x46_hbm, idx_hbm, masked_hbm,
             cls_v, idx_v, tile_v, out_v, sem):
    wid = lax.axis_index("s") * NC + lax.axis_index("c")
    base = wid * BPW
    pltpu.sync_copy(cls_hbm.at[pl.ds(base * C, BPW * C)], cls_v)
    lane = lax.broadcasted_iota(jnp.int32, (L,), 0)
    for g in range(BPW // L):

        def row_body(jj, acc):
            j = g * L + jj

            def chunk(ci, carry):
                vmax, varg = carry
                v = cls_v[pl.ds(j * C + ci * L, L)]
                take = v > vmax
                return (jnp.where(take, v, vmax),
                        jnp.where(take, ci * L + lane, varg))

            vmax, varg = lax.fori_loop(
                0, C // L, chunk,
                (jnp.full((L,), -1.0, jnp.float32),
                 jnp.zeros((L,), jnp.int32)),
            )
            m = jnp.max(vmax)
            c = jnp.min(jnp.where(vmax == m, varg, C))
            t = ((base + j) * C + c) >> 3
            pltpu.sync_copy(x46_hbm.at[t], tile_v)
            k = c & 7
            for mm in range(D // L):
                out_v[pl.ds(j * D + mm * L, L)] = tile_v[k, pl.ds(mm * L, L)]
            return jnp.where(lane == jj, c, acc)

        acc = lax.fori_loop(0, L, row_body, jnp.zeros((L,), jnp.int32))
        idx_v[pl.ds(g * L, L)] = acc
    pltpu.sync_copy(idx_v, idx_hbm.at[pl.ds(base, BPW)])
    pltpu.sync_copy(out_v, masked_hbm.at[pl.ds(base * D, BPW * D)])


def kernel(x):
    cls = _classes(x)
    idx, masked = _sc_pick(cls.reshape(B * C), x.reshape(NTILE, 8, D))
    return masked.reshape(B, 1, D), idx, cls.reshape(B, C)
